# Initial kernel scaffold; baseline (speedup 1.0000x reference)
import jax
import jax.numpy as jnp
from jax.experimental import pallas as pl


def kernel(x_nchw, w_oihw, bias, gamma, beta):
    raise NotImplementedError("write your pallas kernel here")


# speedup vs baseline: 25.5791x; 25.5791x over previous
import jax
import jax.numpy as jnp
from jax.experimental import pallas as pl
from jax.experimental.pallas import tpu as pltpu


def _conv_stats_kernel(nrows, KH, q, xc_ref, wt_ref, b_ref, y_ref, st_ref):
    Hq = xc_ref.shape[2] - q
    Wq = xc_ref.shape[3] - q
    u = {}
    for pi in range(2):
        for io in range(q + 1):
            for joff in range(q + 1):
                sl = xc_ref[pi, 0, io:io + Hq, joff:joff + Wq, :]
                u[(pi, io, joff)] = sl.reshape(nrows, xc_ref.shape[4])
    s = None
    ss = None
    bias = b_ref[...]
    for di in range(2):
        for dj in range(2):
            acc = None
            for kh in range(KH):
                pi = (di + kh) % 2
                io = (di + kh) // 2
                for joff in range(q + 1):
                    d = jnp.dot(u[(pi, io, joff)], wt_ref[dj, kh, joff],
                                preferred_element_type=jnp.float32)
                    acc = d if acc is None else acc + d
            acc = acc + bias
            y_ref[di * 2 + dj, 0] = acc.astype(jnp.bfloat16)
            ps = jnp.sum(acc, axis=0, keepdims=True)
            pss = jnp.sum(acc * acc, axis=0, keepdims=True)
            s = ps if s is None else s + ps
            ss = pss if ss is None else ss + pss
    st_ref[0] = jnp.concatenate([s, ss], axis=0)


def _bn_pool_relu_kernel(y_ref, sc_ref, sh_ref, o_ref):
    sc = sc_ref[...]
    sh = sh_ref[...]
    n0 = y_ref[0, 0].astype(jnp.float32) * sc + sh
    n1 = y_ref[1, 0].astype(jnp.float32) * sc + sh
    n2 = y_ref[2, 0].astype(jnp.float32) * sc + sh
    n3 = y_ref[3, 0].astype(jnp.float32) * sc + sh
    pooled = jnp.maximum(jnp.maximum(n0, n1), jnp.maximum(n2, n3))
    o_ref[0] = jnp.maximum(pooled, 0.0)


def kernel(x_nchw, w_oihw, bias, gamma, beta):
    eps = 1e-5
    N, Cin, H, W = x_nchw.shape
    Cout, Cin2, KH, KW = w_oihw.shape
    assert Cin2 == Cin and KH == KW and KH % 2 == 1
    assert H % 2 == 0 and W % 2 == 0
    q = KH // 2
    Hh, Wh = H // 2, W // 2
    nrows = Hh * Wh
    L = 2 * Cin

    xb = x_nchw.astype(jnp.bfloat16)
    x_nhwc = jnp.transpose(xb, (0, 2, 3, 1))
    xpad = jnp.pad(x_nhwc, ((0, 0), (q, q), (q, q), (0, 0)))
    xr = xpad.reshape(N, Hh + q, 2, Wh + q, 2, Cin)
    xc = jnp.transpose(xr, (2, 0, 1, 3, 4, 5)).reshape(2, N, Hh + q, Wh + q, L)

    wt = jnp.transpose(w_oihw, (2, 3, 1, 0)).astype(jnp.float32)
    Wt = jnp.zeros((2, KH, q + 1, 2, Cin, Cout), jnp.float32)
    for dj in range(2):
        for kh in range(KH):
            for kw in range(KW):
                Wt = Wt.at[dj, kh, (dj + kw) // 2, (dj + kw) % 2].set(wt[kh, kw])
    Wt = Wt.reshape(2, KH, q + 1, L, Cout).astype(jnp.bfloat16)
    b2 = bias.astype(jnp.float32).reshape(1, Cout)

    cparams = pltpu.CompilerParams(
        dimension_semantics=("parallel",),
        vmem_limit_bytes=64 * 1024 * 1024)

    import functools
    y, st = pl.pallas_call(
        functools.partial(_conv_stats_kernel, nrows, KH, q),
        out_shape=(jax.ShapeDtypeStruct((4, N, nrows, Cout), jnp.bfloat16),
                   jax.ShapeDtypeStruct((N, 2, Cout), jnp.float32)),
        grid=(N,),
        in_specs=[
            pl.BlockSpec((2, 1, Hh + q, Wh + q, L), lambda n: (0, n, 0, 0, 0)),
            pl.BlockSpec((2, KH, q + 1, L, Cout), lambda n: (0, 0, 0, 0, 0)),
            pl.BlockSpec((1, Cout), lambda n: (0, 0)),
        ],
        out_specs=(
            pl.BlockSpec((4, 1, nrows, Cout), lambda n: (0, n, 0, 0)),
            pl.BlockSpec((1, 2, Cout), lambda n: (n, 0, 0)),
        ),
        compiler_params=cparams,
    )(xc, Wt, b2)

    stats = jnp.sum(st, axis=0)
    count = jnp.float32(N * H * W)
    mean = stats[0] / count
    var = jnp.maximum(stats[1] / count - mean * mean, 0.0)
    inv = jax.lax.rsqrt(var + eps)
    g = gamma.astype(jnp.float32)
    scale = (inv * g).reshape(1, Cout)
    shift = (beta.astype(jnp.float32) - mean * inv * g).reshape(1, Cout)

    out = pl.pallas_call(
        _bn_pool_relu_kernel,
        out_shape=jax.ShapeDtypeStruct((N, nrows, Cout), jnp.float32),
        grid=(N,),
        in_specs=[
            pl.BlockSpec((4, 1, nrows, Cout), lambda n: (0, n, 0, 0)),
            pl.BlockSpec((1, Cout), lambda n: (0, 0)),
            pl.BlockSpec((1, Cout), lambda n: (0, 0)),
        ],
        out_specs=pl.BlockSpec((1, nrows, Cout), lambda n: (n, 0, 0)),
        compiler_params=cparams,
    )(y, scale, shift)

    out = out.reshape(N, Hh, Wh, Cout)
    return jnp.transpose(out, (0, 3, 1, 2))
